# baseline (speedup 1.0000x reference)
import functools

import jax
import jax.numpy as jnp
from jax import lax
from jax.experimental import pallas as pl
from jax.experimental.pallas import tpu as pltpu
from jax.experimental.pallas import tpu_sc as plsc

N = 10000
M = 10240
E = 320000
IN_DIM = 128
HID = 256
OUT_DIM = 40
EPS = 1e-5

CH = 96
NSUB = 16
ROWS_PER_SUB = M // NSUB
E_SUB = E // NSUB
CHUNKS_P = 216
E_SUB_P = CHUNKS_P * CH
BLK = 8
NBLK = CHUNKS_P // BLK
PAD_ROW = M - 1


def _mesh():
    return plsc.VectorSubcoreMesh(core_axis_name="c", subcore_axis_name="s",
                                  num_cores=1)


def _deg_body(dst3_hbm, zeros_hbm, parts_hbm, didx, ones_v, acc):
    s = lax.axis_index("s")
    onerow = jnp.where(lax.iota(jnp.int32, 16) == 0, 1.0, 0.0).astype(jnp.float32)

    def init_ones(i, carry):
        ones_v[i] = onerow
        return carry

    lax.fori_loop(0, CH, init_ones, 0)
    pltpu.sync_copy(zeros_hbm.at[pl.ds(s * ROWS_PER_SUB, ROWS_PER_SUB)],
                    acc.at[pl.ds(s * ROWS_PER_SUB, ROWS_PER_SUB)])
    pltpu.sync_copy(dst3_hbm.at[s], didx)
    plsc.subcore_barrier()

    def body(i, carry):
        pltpu.sync_copy(ones_v, acc.at[didx.at[i]], add=True)
        return carry

    lax.fori_loop(0, CHUNKS_P, body, 0)
    plsc.subcore_barrier()
    pltpu.sync_copy(acc.at[pl.ds(s * ROWS_PER_SUB, ROWS_PER_SUB)],
                    parts_hbm.at[pl.ds(s * ROWS_PER_SUB, ROWS_PER_SUB)])


@functools.lru_cache(maxsize=None)
def _make_deg():
    return functools.partial(
        pl.kernel,
        mesh=_mesh(),
        out_type=jax.ShapeDtypeStruct((M, 16), jnp.float32),
        scratch_types=[
            pltpu.VMEM((CHUNKS_P, CH), jnp.int32),
            pltpu.VMEM((CH, 16), jnp.float32),
            pltpu.VMEM_SHARED((M, 16), jnp.float32),
        ],
    )(_deg_body)


@functools.lru_cache(maxsize=None)
def _make_agg(npass):
    @functools.partial(
        pl.kernel,
        mesh=_mesh(),
        out_type=[jax.ShapeDtypeStruct((M, 128), jnp.float32)] * npass,
        scratch_types=[
            pltpu.VMEM((BLK, CH), jnp.int32),
            pltpu.VMEM((BLK, CH), jnp.int32),
            pltpu.VMEM((CH, 128), jnp.float32),
            pltpu.VMEM((CH, 128), jnp.float32),
            pltpu.VMEM_SHARED((M, 128), jnp.float32),
            pltpu.SemaphoreType.DMA,
            pltpu.SemaphoreType.DMA,
            pltpu.SemaphoreType.DMA,
            pltpu.SemaphoreType.DMA,
        ],
    )
    def agg(src3_hbm, dst3_hbm, *refs):
        g_hbms = refs[:npass]
        out_hbms = refs[npass:2 * npass]
        sidx, didx, buf0, buf1, acc, gsem0, gsem1, ssem0, ssem1 = \
            refs[2 * npass:]
        s = lax.axis_index("s")
        bufs = (buf0, buf1)
        gsems = (gsem0, gsem1)
        ssems = (ssem0, ssem1)

        def do_pass(g_hbm, out_hbm):
            pltpu.sync_copy(g_hbm.at[pl.ds(s * ROWS_PER_SUB, ROWS_PER_SUB)],
                            acc.at[pl.ds(s * ROWS_PER_SUB, ROWS_PER_SUB)])
            plsc.subcore_barrier()

            def drain_tail():
                pltpu.make_async_copy(buf0, acc.at[didx.at[BLK - 2]],
                                      ssem0).wait()
                pltpu.make_async_copy(buf1, acc.at[didx.at[BLK - 1]],
                                      ssem1).wait()

            def body(b, carry):
                @pl.when(b > 0)
                def _():
                    drain_tail()

                pltpu.sync_copy(src3_hbm.at[s, pl.ds(b * BLK, BLK)], sidx)
                pltpu.sync_copy(dst3_hbm.at[s, pl.ds(b * BLK, BLK)], didx)
                pltpu.async_copy(g_hbm.at[sidx.at[0]], buf0, gsem0)
                pltpu.async_copy(g_hbm.at[sidx.at[1]], buf1, gsem1)
                for j in range(BLK):
                    p = j % 2
                    pltpu.make_async_copy(g_hbm.at[sidx.at[j]], bufs[p],
                                          gsems[p]).wait()
                    pltpu.async_copy(bufs[p], acc.at[didx.at[j]], ssems[p],
                                     add=True)
                    if j + 2 < BLK:
                        pltpu.make_async_copy(bufs[p], acc.at[didx.at[j]],
                                              ssems[p]).wait()
                        pltpu.async_copy(g_hbm.at[sidx.at[j + 2]], bufs[p],
                                         gsems[p])
                return carry

            lax.fori_loop(0, NBLK, body, 0)
            drain_tail()
            plsc.subcore_barrier()
            pltpu.sync_copy(acc.at[pl.ds(s * ROWS_PER_SUB, ROWS_PER_SUB)],
                            out_hbm.at[pl.ds(s * ROWS_PER_SUB, ROWS_PER_SUB)])

        for p in range(npass):
            do_pass(g_hbms[p], out_hbms[p])

    return agg


BR = 1024
NBR = M // BR
_F32 = jnp.float32


def _rowspec(cols):
    return pl.BlockSpec((BR, cols), lambda j: (j, 0))


def _fullspec(shape):
    return pl.BlockSpec(shape, lambda j: tuple(0 for _ in shape))


def _dinv_body(p_ref, dinv_ref):
    dinv_ref[...] = lax.rsqrt(p_ref[:, 0:1] + 1.0)


def _dinv_call(parts):
    return pl.pallas_call(
        _dinv_body, out_shape=jax.ShapeDtypeStruct((M, 1), _F32))(parts)


def _f0_body(x_ref, w_ref, dinv_ref, g0_ref, g1_ref):
    h = jnp.dot(x_ref[...], w_ref[...], preferred_element_type=_F32,
                precision=lax.Precision.HIGHEST)
    g = h * dinv_ref[...]
    g0_ref[...] = g[:, :128]
    g1_ref[...] = g[:, 128:]


def _f0_call(xp, W0, dinv):
    return pl.pallas_call(
        _f0_body,
        grid=(NBR,),
        in_specs=[_rowspec(IN_DIM), _fullspec((IN_DIM, HID)), _rowspec(1)],
        out_specs=[_rowspec(128)] * 2,
        out_shape=[jax.ShapeDtypeStruct((M, 128), _F32)] * 2,
    )(xp, W0, dinv)


def _stats_body(a0_ref, a1_ref, dinv_ref, b_ref, s1_ref, s2_ref):
    j = pl.program_id(0)
    d = dinv_ref[...]
    rows = lax.broadcasted_iota(jnp.int32, (BR, 1), 0) + j * BR
    mask = rows < N
    t0 = a0_ref[...] * d + b_ref[:, :128]
    t1 = a1_ref[...] * d + b_ref[:, 128:]
    t = jnp.concatenate([t0, t1], axis=1)
    tm = jnp.where(mask, t, 0.0)
    ps1 = jnp.sum(tm, axis=0, keepdims=True)
    ps2 = jnp.sum(tm * tm, axis=0, keepdims=True)

    @pl.when(j == 0)
    def _():
        s1_ref[...] = ps1
        s2_ref[...] = ps2

    @pl.when(j > 0)
    def _():
        s1_ref[...] += ps1
        s2_ref[...] += ps2


def _stats_call(a0, a1, dinv, b):
    return pl.pallas_call(
        _stats_body,
        grid=(NBR,),
        in_specs=[_rowspec(128), _rowspec(128), _rowspec(1),
                  _fullspec((1, HID))],
        out_specs=[_fullspec((1, HID))] * 2,
        out_shape=[jax.ShapeDtypeStruct((1, HID), _F32)] * 2,
    )(a0, a1, dinv, b)


def _apply_body(a0_ref, a1_ref, dinv_ref, b_ref, gam_ref, bet_ref, s1_ref,
                s2_ref, w_ref, *g_refs):
    d = dinv_ref[...]
    inv_n = 1.0 / N
    mean = s1_ref[...] * inv_n
    var = s2_ref[...] * inv_n - mean * mean
    rstd = lax.rsqrt(var + EPS)
    g = None
    for q, a_ref in enumerate((a0_ref, a1_ref)):
        lo = q * 128
        t = a_ref[...] * d + b_ref[:, lo:lo + 128]
        y = (t - mean[:, lo:lo + 128]) * rstd[:, lo:lo + 128]
        y = y * gam_ref[:, lo:lo + 128] + bet_ref[:, lo:lo + 128]
        y = jnp.maximum(y, 0.0)
        part = jnp.dot(y, w_ref[lo:lo + 128, :],
                       preferred_element_type=_F32,
                       precision=lax.Precision.HIGHEST)
        g = part if g is None else g + part
    g = g * d
    w = g.shape[1] // len(g_refs)
    for q, ref in enumerate(g_refs):
        ref[...] = g[:, q * w:(q + 1) * w]


def _apply_call(a0, a1, dinv, b, gam, bet, s1, s2, W, nout):
    wout = W.shape[1] // nout
    return pl.pallas_call(
        _apply_body,
        grid=(NBR,),
        in_specs=[_rowspec(128), _rowspec(128), _rowspec(1),
                  _fullspec((1, HID)), _fullspec((1, HID)),
                  _fullspec((1, HID)), _fullspec((1, HID)),
                  _fullspec((1, HID)), _fullspec((HID, W.shape[1]))],
        out_specs=[_rowspec(wout)] * nout,
        out_shape=[jax.ShapeDtypeStruct((M, wout), _F32)] * nout,
    )(a0, a1, dinv, b, gam, bet, s1, s2, W)


def _f5_body(c_ref, dinv_ref, bl_ref, out_ref):
    t = c_ref[...] * dinv_ref[...] + bl_ref[...]
    colmask = lax.broadcasted_iota(jnp.int32, (1, 128), 1) < OUT_DIM
    mx = jnp.max(jnp.where(colmask, t, -jnp.inf), axis=1, keepdims=True)
    se = jnp.sum(jnp.where(colmask, jnp.exp(t - mx), 0.0), axis=1,
                 keepdims=True)
    out_ref[...] = t - mx - jnp.log(se)


def _f5_call(c5, dinv, bl_pad):
    return pl.pallas_call(
        _f5_body,
        grid=(NBR,),
        in_specs=[_rowspec(128), _rowspec(1), _fullspec((1, 128))],
        out_specs=_rowspec(128),
        out_shape=jax.ShapeDtypeStruct((M, 128), _F32),
    )(c5, dinv, bl_pad)


def kernel(x, adj_t, W0, b0, Wm, bm, Wl, bl, gamma, beta):
    f32 = jnp.float32
    ei = adj_t.astype(jnp.int32)
    npad = E_SUB_P - E_SUB
    src3 = jnp.pad(ei[0].reshape(NSUB, E_SUB), ((0, 0), (0, npad))
                   ).reshape(NSUB, CHUNKS_P, CH)
    dst3 = jnp.pad(ei[1].reshape(NSUB, E_SUB), ((0, 0), (0, npad)),
                   constant_values=PAD_ROW).reshape(NSUB, CHUNKS_P, CH)
    xp = jnp.zeros((M, IN_DIM), f32).at[:N].set(x)
    zeros16 = jnp.zeros((M, 16), f32)
    Wl_pad = jnp.zeros((HID, 128), f32).at[:, :OUT_DIM].set(Wl)
    bl_pad = jnp.zeros((1, 128), f32).at[:, :OUT_DIM].set(bl)
    b0r = b0.reshape(1, HID)
    bmr = bm.reshape(1, HID)
    gam = gamma.reshape(1, HID)
    bet = beta.reshape(1, HID)

    parts = _make_deg()(dst3, zeros16)
    dinv = _dinv_call(parts)

    gs = _f0_call(xp, W0, dinv)
    for bcur in (b0r, bmr, bmr):
        a0, a1 = _make_agg(2)(src3, dst3, *gs)
        s1, s2 = _stats_call(a0, a1, dinv, bcur)
        gs = _apply_call(a0, a1, dinv, bcur, gam, bet, s1, s2, Wm, 2)
    a0, a1 = _make_agg(2)(src3, dst3, *gs)
    s1, s2 = _stats_call(a0, a1, dinv, bmr)
    g5, = _apply_call(a0, a1, dinv, bmr, gam, bet, s1, s2, Wl_pad, 1)
    c5, = _make_agg(1)(src3, dst3, g5)
    out = _f5_call(c5, dinv, bl_pad)
    return out[:N, :OUT_DIM]


# speedup vs baseline: 1.3336x; 1.3336x over previous
import functools

import jax
import jax.numpy as jnp
from jax import lax
from jax.experimental import pallas as pl
from jax.experimental.pallas import tpu as pltpu
from jax.experimental.pallas import tpu_sc as plsc

N = 10000
M = 10240
E = 320000
IN_DIM = 128
HID = 256
OUT_DIM = 40
EPS = 1e-5

CH = 80
NSUB = 16
ROWS_PER_SUB = M // NSUB
E_SUB = E // NSUB
CHUNKS_P = 256
E_SUB_P = CHUNKS_P * CH
BLK = 32
NBLK = CHUNKS_P // BLK
PAD_ROW = M - 1


def _mesh():
    return plsc.VectorSubcoreMesh(core_axis_name="c", subcore_axis_name="s",
                                  num_cores=1)


def _deg_body(dst3_hbm, zeros_hbm, parts_hbm, didx, ones_v, acc):
    s = lax.axis_index("s")
    onerow = jnp.where(lax.iota(jnp.int32, 16) == 0, 1.0, 0.0).astype(jnp.float32)

    def init_ones(i, carry):
        ones_v[i] = onerow
        return carry

    lax.fori_loop(0, CH, init_ones, 0)
    pltpu.sync_copy(zeros_hbm.at[pl.ds(s * ROWS_PER_SUB, ROWS_PER_SUB)],
                    acc.at[pl.ds(s * ROWS_PER_SUB, ROWS_PER_SUB)])
    pltpu.sync_copy(dst3_hbm.at[s], didx)
    plsc.subcore_barrier()

    def body(i, carry):
        pltpu.sync_copy(ones_v, acc.at[didx.at[i]], add=True)
        return carry

    lax.fori_loop(0, CHUNKS_P, body, 0)
    plsc.subcore_barrier()
    pltpu.sync_copy(acc.at[pl.ds(s * ROWS_PER_SUB, ROWS_PER_SUB)],
                    parts_hbm.at[pl.ds(s * ROWS_PER_SUB, ROWS_PER_SUB)])


@functools.lru_cache(maxsize=None)
def _make_deg():
    return functools.partial(
        pl.kernel,
        mesh=_mesh(),
        out_type=jax.ShapeDtypeStruct((M, 16), jnp.float32),
        scratch_types=[
            pltpu.VMEM((CHUNKS_P, CH), jnp.int32),
            pltpu.VMEM((CH, 16), jnp.float32),
            pltpu.VMEM_SHARED((M, 16), jnp.float32),
        ],
    )(_deg_body)


@functools.lru_cache(maxsize=None)
def _make_agg(npass):
    @functools.partial(
        pl.kernel,
        mesh=_mesh(),
        out_type=[jax.ShapeDtypeStruct((M, 128), jnp.float32)] * npass,
        scratch_types=[
            pltpu.VMEM((BLK, CH), jnp.int32),
            pltpu.VMEM((BLK, CH), jnp.int32),
            pltpu.VMEM((CH, 128), jnp.float32),
            pltpu.VMEM((CH, 128), jnp.float32),
            pltpu.VMEM_SHARED((M, 128), jnp.float32),
            pltpu.SemaphoreType.DMA,
            pltpu.SemaphoreType.DMA,
            pltpu.SemaphoreType.DMA,
            pltpu.SemaphoreType.DMA,
        ],
    )
    def agg(src3_hbm, dst3_hbm, *refs):
        g_hbms = refs[:npass]
        out_hbms = refs[npass:2 * npass]
        sidx, didx, buf0, buf1, acc, gsem0, gsem1, ssem0, ssem1 = \
            refs[2 * npass:]
        s = lax.axis_index("s")
        bufs = (buf0, buf1)
        gsems = (gsem0, gsem1)
        ssems = (ssem0, ssem1)

        def do_pass(g_hbm, out_hbm):
            pltpu.sync_copy(g_hbm.at[pl.ds(s * ROWS_PER_SUB, ROWS_PER_SUB)],
                            acc.at[pl.ds(s * ROWS_PER_SUB, ROWS_PER_SUB)])
            plsc.subcore_barrier()

            def drain_tail():
                pltpu.make_async_copy(buf0, acc.at[didx.at[BLK - 2]],
                                      ssem0).wait()
                pltpu.make_async_copy(buf1, acc.at[didx.at[BLK - 1]],
                                      ssem1).wait()

            def body(b, carry):
                @pl.when(b > 0)
                def _():
                    drain_tail()

                pltpu.sync_copy(src3_hbm.at[s, pl.ds(b * BLK, BLK)], sidx)
                pltpu.sync_copy(dst3_hbm.at[s, pl.ds(b * BLK, BLK)], didx)
                pltpu.async_copy(g_hbm.at[sidx.at[0]], buf0, gsem0)
                pltpu.async_copy(g_hbm.at[sidx.at[1]], buf1, gsem1)
                for j in range(BLK):
                    p = j % 2
                    pltpu.make_async_copy(g_hbm.at[sidx.at[j]], bufs[p],
                                          gsems[p]).wait()
                    pltpu.async_copy(bufs[p], acc.at[didx.at[j]], ssems[p],
                                     add=True)
                    if j + 2 < BLK:
                        pltpu.make_async_copy(bufs[p], acc.at[didx.at[j]],
                                              ssems[p]).wait()
                        pltpu.async_copy(g_hbm.at[sidx.at[j + 2]], bufs[p],
                                         gsems[p])
                return carry

            lax.fori_loop(0, NBLK, body, 0)
            drain_tail()
            plsc.subcore_barrier()
            pltpu.sync_copy(acc.at[pl.ds(s * ROWS_PER_SUB, ROWS_PER_SUB)],
                            out_hbm.at[pl.ds(s * ROWS_PER_SUB, ROWS_PER_SUB)])

        for p in range(npass):
            do_pass(g_hbms[p], out_hbms[p])

    return agg


BR = 1024
NBR = M // BR
_F32 = jnp.float32


def _rowspec(cols):
    return pl.BlockSpec((BR, cols), lambda j: (j, 0))


def _fullspec(shape):
    return pl.BlockSpec(shape, lambda j: tuple(0 for _ in shape))


def _dinv_body(p_ref, dinv_ref):
    dinv_ref[...] = lax.rsqrt(p_ref[:, 0:1] + 1.0)


def _dinv_call(parts):
    return pl.pallas_call(
        _dinv_body, out_shape=jax.ShapeDtypeStruct((M, 1), _F32))(parts)


def _f0_body(x_ref, w_ref, dinv_ref, g0_ref, g1_ref):
    h = jnp.dot(x_ref[...], w_ref[...], preferred_element_type=_F32,
                precision=lax.Precision.HIGHEST)
    g = h * dinv_ref[...]
    g0_ref[...] = g[:, :128]
    g1_ref[...] = g[:, 128:]


def _f0_call(xp, W0, dinv):
    return pl.pallas_call(
        _f0_body,
        grid=(NBR,),
        in_specs=[_rowspec(IN_DIM), _fullspec((IN_DIM, HID)), _rowspec(1)],
        out_specs=[_rowspec(128)] * 2,
        out_shape=[jax.ShapeDtypeStruct((M, 128), _F32)] * 2,
    )(xp, W0, dinv)


def _stats_body(a0_ref, a1_ref, dinv_ref, b_ref, s1_ref, s2_ref):
    j = pl.program_id(0)
    d = dinv_ref[...]
    rows = lax.broadcasted_iota(jnp.int32, (BR, 1), 0) + j * BR
    mask = rows < N
    t0 = a0_ref[...] * d + b_ref[:, :128]
    t1 = a1_ref[...] * d + b_ref[:, 128:]
    t = jnp.concatenate([t0, t1], axis=1)
    tm = jnp.where(mask, t, 0.0)
    ps1 = jnp.sum(tm, axis=0, keepdims=True)
    ps2 = jnp.sum(tm * tm, axis=0, keepdims=True)

    @pl.when(j == 0)
    def _():
        s1_ref[...] = ps1
        s2_ref[...] = ps2

    @pl.when(j > 0)
    def _():
        s1_ref[...] += ps1
        s2_ref[...] += ps2


def _stats_call(a0, a1, dinv, b):
    return pl.pallas_call(
        _stats_body,
        grid=(NBR,),
        in_specs=[_rowspec(128), _rowspec(128), _rowspec(1),
                  _fullspec((1, HID))],
        out_specs=[_fullspec((1, HID))] * 2,
        out_shape=[jax.ShapeDtypeStruct((1, HID), _F32)] * 2,
    )(a0, a1, dinv, b)


def _apply_body(a0_ref, a1_ref, dinv_ref, b_ref, gam_ref, bet_ref, s1_ref,
                s2_ref, w_ref, *g_refs):
    d = dinv_ref[...]
    inv_n = 1.0 / N
    mean = s1_ref[...] * inv_n
    var = s2_ref[...] * inv_n - mean * mean
    rstd = lax.rsqrt(var + EPS)
    g = None
    for q, a_ref in enumerate((a0_ref, a1_ref)):
        lo = q * 128
        t = a_ref[...] * d + b_ref[:, lo:lo + 128]
        y = (t - mean[:, lo:lo + 128]) * rstd[:, lo:lo + 128]
        y = y * gam_ref[:, lo:lo + 128] + bet_ref[:, lo:lo + 128]
        y = jnp.maximum(y, 0.0)
        part = jnp.dot(y, w_ref[lo:lo + 128, :],
                       preferred_element_type=_F32,
                       precision=lax.Precision.HIGHEST)
        g = part if g is None else g + part
    g = g * d
    w = g.shape[1] // len(g_refs)
    for q, ref in enumerate(g_refs):
        ref[...] = g[:, q * w:(q + 1) * w]


def _apply_call(a0, a1, dinv, b, gam, bet, s1, s2, W, nout):
    wout = W.shape[1] // nout
    return pl.pallas_call(
        _apply_body,
        grid=(NBR,),
        in_specs=[_rowspec(128), _rowspec(128), _rowspec(1),
                  _fullspec((1, HID)), _fullspec((1, HID)),
                  _fullspec((1, HID)), _fullspec((1, HID)),
                  _fullspec((1, HID)), _fullspec((HID, W.shape[1]))],
        out_specs=[_rowspec(wout)] * nout,
        out_shape=[jax.ShapeDtypeStruct((M, wout), _F32)] * nout,
    )(a0, a1, dinv, b, gam, bet, s1, s2, W)


def _f5_body(c_ref, dinv_ref, bl_ref, out_ref):
    t = c_ref[...] * dinv_ref[...] + bl_ref[...]
    colmask = lax.broadcasted_iota(jnp.int32, (1, 128), 1) < OUT_DIM
    mx = jnp.max(jnp.where(colmask, t, -jnp.inf), axis=1, keepdims=True)
    se = jnp.sum(jnp.where(colmask, jnp.exp(t - mx), 0.0), axis=1,
                 keepdims=True)
    out_ref[...] = t - mx - jnp.log(se)


def _f5_call(c5, dinv, bl_pad):
    return pl.pallas_call(
        _f5_body,
        grid=(NBR,),
        in_specs=[_rowspec(128), _rowspec(1), _fullspec((1, 128))],
        out_specs=_rowspec(128),
        out_shape=jax.ShapeDtypeStruct((M, 128), _F32),
    )(c5, dinv, bl_pad)


def kernel(x, adj_t, W0, b0, Wm, bm, Wl, bl, gamma, beta):
    f32 = jnp.float32
    ei = adj_t.astype(jnp.int32)
    npad = E_SUB_P - E_SUB
    src3 = jnp.pad(ei[0].reshape(NSUB, E_SUB), ((0, 0), (0, npad))
                   ).reshape(NSUB, CHUNKS_P, CH)
    dst3 = jnp.pad(ei[1].reshape(NSUB, E_SUB), ((0, 0), (0, npad)),
                   constant_values=PAD_ROW).reshape(NSUB, CHUNKS_P, CH)
    xp = jnp.zeros((M, IN_DIM), f32).at[:N].set(x)
    zeros16 = jnp.zeros((M, 16), f32)
    Wl_pad = jnp.zeros((HID, 128), f32).at[:, :OUT_DIM].set(Wl)
    bl_pad = jnp.zeros((1, 128), f32).at[:, :OUT_DIM].set(bl)
    b0r = b0.reshape(1, HID)
    bmr = bm.reshape(1, HID)
    gam = gamma.reshape(1, HID)
    bet = beta.reshape(1, HID)

    parts = _make_deg()(dst3, zeros16)
    dinv = _dinv_call(parts)

    gs = _f0_call(xp, W0, dinv)
    for bcur in (b0r, bmr, bmr):
        a0, a1 = _make_agg(2)(src3, dst3, *gs)
        s1, s2 = _stats_call(a0, a1, dinv, bcur)
        gs = _apply_call(a0, a1, dinv, bcur, gam, bet, s1, s2, Wm, 2)
    a0, a1 = _make_agg(2)(src3, dst3, *gs)
    s1, s2 = _stats_call(a0, a1, dinv, bmr)
    g5, = _apply_call(a0, a1, dinv, bmr, gam, bet, s1, s2, Wl_pad, 1)
    c5, = _make_agg(1)(src3, dst3, g5)
    out = _f5_call(c5, dinv, bl_pad)
    return out[:N, :OUT_DIM]
